# Initial kernel scaffold; baseline (speedup 1.0000x reference)
import functools
import math

import jax
import jax.numpy as jnp
import numpy as np
from jax import lax
from jax.experimental import pallas as pl
from jax.experimental.pallas import tpu as pltpu
from jax.experimental.pallas import tpu_sc as plsc

N_NODES = 5000
D = 128
N_EDGES = 320000
NPAD = 5120
LANES = 128
EPAD = 323584
NB = 2528
NW = 32
NBT = NB // NW
ROWS_PER_TILE = NPAD // 16


def _positional_encoding(n, d):
    position = np.arange(0, n, dtype=np.float32)[:, None]
    div_term = np.exp(np.arange(0, d, 2).astype(np.float32) * -(math.log(10000.0) / d))
    enc = np.zeros((n, d), dtype=np.float32)
    enc[:, 0::2] = np.sin(position * div_term)
    enc[:, 1::2] = np.cos(position * div_term)
    return jnp.asarray(enc)


_MESH = plsc.VectorSubcoreMesh(core_axis_name="c", subcore_axis_name="s")


@functools.partial(
    pl.kernel,
    out_type=jax.ShapeDtypeStruct((2, NPAD, 16), jnp.float32),
    mesh=_MESH,
    scratch_types=[
        pltpu.VMEM_SHARED((NPAD, 16), jnp.float32),
        pltpu.VMEM((NBT, LANES), jnp.int32),
        pltpu.VMEM((LANES, 16), jnp.float32),
        pltpu.VMEM((ROWS_PER_TILE, 16), jnp.float32),
        pltpu.SemaphoreType.DMA,
    ],
)
def _sc_degree(colb_hbm, ones_hbm, z_hbm, out_hbm, acc, coli, ones_v, z_v, sem):
    c = lax.axis_index("c")
    s = lax.axis_index("s")
    wid = s * 2 + c
    pltpu.sync_copy(colb_hbm.at[pl.ds(wid * NBT, NBT)], coli)
    pltpu.sync_copy(ones_hbm, ones_v)
    pltpu.sync_copy(z_hbm, z_v)
    pltpu.sync_copy(z_v, acc.at[pl.ds(s * ROWS_PER_TILE, ROWS_PER_TILE)])
    plsc.subcore_barrier()

    def body(j, carry):
        pltpu.sync_copy(ones_v, acc.at[coli.at[j]], add=True)
        return carry

    lax.fori_loop(0, NBT, body, 0)
    plsc.subcore_barrier()
    pltpu.sync_copy(
        acc.at[pl.ds(s * ROWS_PER_TILE, ROWS_PER_TILE)],
        out_hbm.at[c, pl.ds(s * ROWS_PER_TILE, ROWS_PER_TILE)],
    )


@functools.partial(
    pl.kernel,
    out_type=jax.ShapeDtypeStruct((2, NPAD, D), jnp.float32),
    mesh=_MESH,
    scratch_types=[
        pltpu.VMEM_SHARED((NPAD, D), jnp.float32),
        pltpu.VMEM((NBT, LANES), jnp.int32),
        pltpu.VMEM((NBT, LANES), jnp.int32),
        pltpu.VMEM((80, D), jnp.float32),
        pltpu.VMEM((LANES, D), jnp.float32),
        pltpu.SemaphoreType.DMA,
    ],
)
def _sc_messages(xs_hbm, rowb_hbm, colb_hbm, z_hbm, out_hbm,
                 acc, rowi, coli, z_v, rbuf, sem):
    c = lax.axis_index("c")
    s = lax.axis_index("s")
    wid = s * 2 + c
    pltpu.sync_copy(rowb_hbm.at[pl.ds(wid * NBT, NBT)], rowi)
    pltpu.sync_copy(colb_hbm.at[pl.ds(wid * NBT, NBT)], coli)
    pltpu.sync_copy(z_hbm, z_v)
    for k in range(ROWS_PER_TILE // 80):
        pltpu.sync_copy(z_v, acc.at[pl.ds(s * ROWS_PER_TILE + k * 80, 80)])
    plsc.subcore_barrier()

    def body(j, carry):
        pltpu.async_copy(xs_hbm.at[rowi.at[j]], rbuf, sem).wait()
        pltpu.sync_copy(rbuf, acc.at[coli.at[j]], add=True)
        return carry

    lax.fori_loop(0, NBT, body, 0)
    plsc.subcore_barrier()
    pltpu.sync_copy(
        acc.at[pl.ds(s * ROWS_PER_TILE, ROWS_PER_TILE)],
        out_hbm.at[c, pl.ds(s * ROWS_PER_TILE, ROWS_PER_TILE)],
    )


def _tc_prep_body(x_ref, pe_ref, w1_ref, dp_ref, xs_ref, dis_ref):
    deg = dp_ref[0, : N_NODES, 0:1] + dp_ref[1, : N_NODES, 0:1] + 1.0
    dis = lax.rsqrt(deg)
    xw = jnp.dot(x_ref[...] + pe_ref[...], w1_ref[...],
                 preferred_element_type=jnp.float32)
    xs_ref[...] = dis * xw
    dis_ref[...] = dis


def _tc_mid_body(m_ref, xs_ref, dis_ref, b1_ref, g_ref, bt_ref, w2_ref, out_ref):
    msum = m_ref[0, : N_NODES, :] + m_ref[1, : N_NODES, :]
    h = dis_ref[...] * (msum + xs_ref[...]) + b1_ref[...]
    mu = jnp.mean(h, axis=0, keepdims=True)
    var = jnp.mean((h - mu) * (h - mu), axis=0, keepdims=True)
    hn = g_ref[...] * (h - mu) * lax.rsqrt(var + 1e-5) + bt_ref[...]
    r = jnp.maximum(hn, 0.0)
    out_ref[...] = dis_ref[...] * jnp.dot(r, w2_ref[...],
                                          preferred_element_type=jnp.float32)


def _tc_final_body(m_ref, xs_ref, dis_ref, b2_ref, out_ref):
    msum = m_ref[0, : N_NODES, :] + m_ref[1, : N_NODES, :]
    out_ref[...] = jnp.maximum(dis_ref[...] * (msum + xs_ref[...]) + b2_ref[...], 0.0)


def kernel(x, edge_index, W1, b1, gamma, beta, W2, b2):
    pe = _positional_encoding(N_NODES, D)
    ei = edge_index.astype(jnp.int32)
    pad = jnp.broadcast_to(
        jnp.array([[0], [NPAD - 1]], dtype=jnp.int32), (2, EPAD - N_EDGES)
    )
    eib = jnp.concatenate([ei, pad], axis=1).reshape(2, NB, LANES)
    rowb = eib[0]
    colb = eib[1]

    ones16 = jnp.ones((LANES, 16), jnp.float32)
    z16 = jnp.zeros((ROWS_PER_TILE, 16), jnp.float32)
    z128 = jnp.zeros((80, D), jnp.float32)

    degparts = _sc_degree(colb, ones16, z16)

    xs1, dis = pl.pallas_call(
        _tc_prep_body,
        out_shape=(
            jax.ShapeDtypeStruct((N_NODES, D), jnp.float32),
            jax.ShapeDtypeStruct((N_NODES, 1), jnp.float32),
        ),
    )(x, pe, W1, degparts)

    m1 = _sc_messages(xs1, rowb, colb, z128)

    xs2 = pl.pallas_call(
        _tc_mid_body,
        out_shape=jax.ShapeDtypeStruct((N_NODES, D), jnp.float32),
    )(m1, xs1, dis, b1.reshape(1, D), gamma.reshape(1, D), beta.reshape(1, D), W2)

    m2 = _sc_messages(xs2, rowb, colb, z128)

    out = pl.pallas_call(
        _tc_final_body,
        out_shape=jax.ShapeDtypeStruct((N_NODES, D), jnp.float32),
    )(m2, xs2, dis, b2.reshape(1, D))
    return out


# speedup vs baseline: 8.3981x; 8.3981x over previous
import functools
import math

import jax
import jax.numpy as jnp
import numpy as np
from jax import lax
from jax.experimental import pallas as pl
from jax.experimental.pallas import tpu as pltpu
from jax.experimental.pallas import tpu_sc as plsc

N_NODES = 5000
D = 128
N_EDGES = 320000
NPAD = 5120
LANES = 128
EPAD = 327680
NB = 2560
NW = 32
NBT = NB // NW
ROWS_PER_TILE = NPAD // 16


def _positional_encoding(n, d):
    position = np.arange(0, n, dtype=np.float32)[:, None]
    div_term = np.exp(np.arange(0, d, 2).astype(np.float32) * -(math.log(10000.0) / d))
    enc = np.zeros((n, d), dtype=np.float32)
    enc[:, 0::2] = np.sin(position * div_term)
    enc[:, 1::2] = np.cos(position * div_term)
    return jnp.asarray(enc)


@functools.cache
def _build_sc_kernels():
    mesh = plsc.VectorSubcoreMesh(
        core_axis_name="c", subcore_axis_name="s", num_cores=2, num_subcores=16
    )

    @functools.partial(
        pl.kernel,
        out_type=jax.ShapeDtypeStruct((2, NPAD, D), jnp.float32),
        mesh=mesh,
        scratch_types=[
            pltpu.VMEM_SHARED((NPAD, D), jnp.float32),
            pltpu.VMEM((NBT, LANES), jnp.int32),
            pltpu.VMEM((LANES, D), jnp.float32),
            pltpu.VMEM((80, D), jnp.float32),
            pltpu.SemaphoreType.DMA,
        ],
    )
    def sc_degree(colb_hbm, ones_hbm, z_hbm, out_hbm, acc, coli, ones_v, z_v, sem):
        c = lax.axis_index("c")
        s = lax.axis_index("s")
        wid = s * 2 + c
        pltpu.sync_copy(colb_hbm.at[pl.ds(wid * NBT, NBT)], coli)
        pltpu.sync_copy(ones_hbm, ones_v)
        pltpu.sync_copy(z_hbm, z_v)
        for k in range(ROWS_PER_TILE // 80):
            pltpu.sync_copy(z_v, acc.at[pl.ds(s * ROWS_PER_TILE + k * 80, 80)])
        plsc.subcore_barrier()

        def body(j, carry):
            pltpu.sync_copy(ones_v, acc.at[coli.at[j]], add=True)
            return carry

        lax.fori_loop(0, NBT, body, 0)
        plsc.subcore_barrier()
        pltpu.sync_copy(
            acc.at[pl.ds(s * ROWS_PER_TILE, ROWS_PER_TILE)],
            out_hbm.at[c, pl.ds(s * ROWS_PER_TILE, ROWS_PER_TILE)],
        )

    @functools.partial(
        pl.kernel,
        out_type=jax.ShapeDtypeStruct((2, NPAD, D), jnp.float32),
        mesh=mesh,
        scratch_types=[
            pltpu.VMEM_SHARED((NPAD, D), jnp.float32),
            pltpu.VMEM((NBT, LANES), jnp.int32),
            pltpu.VMEM((NBT, LANES), jnp.int32),
            pltpu.VMEM((80, D), jnp.float32),
            pltpu.VMEM((LANES, D), jnp.float32),
            pltpu.SemaphoreType.DMA,
        ],
    )
    def sc_messages(xs_hbm, rowb_hbm, colb_hbm, z_hbm, out_hbm,
                    acc, rowi, coli, z_v, rbuf, sem):
        c = lax.axis_index("c")
        s = lax.axis_index("s")
        wid = s * 2 + c
        pltpu.sync_copy(rowb_hbm.at[pl.ds(wid * NBT, NBT)], rowi)
        pltpu.sync_copy(colb_hbm.at[pl.ds(wid * NBT, NBT)], coli)
        pltpu.sync_copy(z_hbm, z_v)
        for k in range(ROWS_PER_TILE // 80):
            pltpu.sync_copy(z_v, acc.at[pl.ds(s * ROWS_PER_TILE + k * 80, 80)])
        plsc.subcore_barrier()

        def body(j, carry):
            pltpu.async_copy(xs_hbm.at[rowi.at[j]], rbuf, sem).wait()
            pltpu.sync_copy(rbuf, acc.at[coli.at[j]], add=True)
            return carry

        lax.fori_loop(0, NBT, body, 0)
        plsc.subcore_barrier()
        pltpu.sync_copy(
            acc.at[pl.ds(s * ROWS_PER_TILE, ROWS_PER_TILE)],
            out_hbm.at[c, pl.ds(s * ROWS_PER_TILE, ROWS_PER_TILE)],
        )

    return sc_degree, sc_messages


def _tc_prep_body(x_ref, pe_ref, w1_ref, dp_ref, xs_ref, dis_ref):
    deg = dp_ref[0, : N_NODES, 0:1] + dp_ref[1, : N_NODES, 0:1] + 1.0
    dis = lax.rsqrt(deg)
    xw = jnp.dot(x_ref[...] + pe_ref[...], w1_ref[...],
                 preferred_element_type=jnp.float32)
    xs_ref[...] = dis * xw
    dis_ref[...] = dis


def _tc_mid_body(m_ref, xs_ref, dis_ref, b1_ref, g_ref, bt_ref, w2_ref, out_ref):
    msum = m_ref[0, : N_NODES, :] + m_ref[1, : N_NODES, :]
    h = dis_ref[...] * (msum + xs_ref[...]) + b1_ref[...]
    mu = jnp.mean(h, axis=0, keepdims=True)
    var = jnp.mean((h - mu) * (h - mu), axis=0, keepdims=True)
    hn = g_ref[...] * (h - mu) * lax.rsqrt(var + 1e-5) + bt_ref[...]
    r = jnp.maximum(hn, 0.0)
    out_ref[...] = dis_ref[...] * jnp.dot(r, w2_ref[...],
                                          preferred_element_type=jnp.float32)


def _tc_final_body(m_ref, xs_ref, dis_ref, b2_ref, out_ref):
    msum = m_ref[0, : N_NODES, :] + m_ref[1, : N_NODES, :]
    out_ref[...] = jnp.maximum(dis_ref[...] * (msum + xs_ref[...]) + b2_ref[...], 0.0)


def kernel(x, edge_index, W1, b1, gamma, beta, W2, b2):
    pe = _positional_encoding(N_NODES, D)
    ei = edge_index.astype(jnp.int32)
    pad = jnp.broadcast_to(
        jnp.array([[0], [NPAD - 1]], dtype=jnp.int32), (2, EPAD - N_EDGES)
    )
    eib = jnp.concatenate([ei, pad], axis=1).reshape(2, NB, LANES)
    rowb = eib[0]
    colb = eib[1]

    ones16 = jnp.ones((LANES, D), jnp.float32)
    z128 = jnp.zeros((80, D), jnp.float32)

    sc_degree, sc_messages = _build_sc_kernels()
    degparts = sc_degree(colb, ones16, z128)

    xs1, dis = pl.pallas_call(
        _tc_prep_body,
        out_shape=(
            jax.ShapeDtypeStruct((N_NODES, D), jnp.float32),
            jax.ShapeDtypeStruct((N_NODES, 1), jnp.float32),
        ),
    )(x, pe, W1, degparts)

    m1 = sc_messages(xs1, rowb, colb, z128)

    xs2 = pl.pallas_call(
        _tc_mid_body,
        out_shape=jax.ShapeDtypeStruct((N_NODES, D), jnp.float32),
    )(m1, xs1, dis, b1.reshape(1, D), gamma.reshape(1, D), beta.reshape(1, D), W2)

    m2 = sc_messages(xs2, rowb, colb, z128)

    out = pl.pallas_call(
        _tc_final_body,
        out_shape=jax.ShapeDtypeStruct((N_NODES, D), jnp.float32),
    )(m2, xs2, dis, b2.reshape(1, D))
    return out


# speedup vs baseline: 9.0433x; 1.0768x over previous
import functools
import math

import jax
import jax.numpy as jnp
import numpy as np
from jax import lax
from jax.experimental import pallas as pl
from jax.experimental.pallas import tpu as pltpu
from jax.experimental.pallas import tpu_sc as plsc

N_NODES = 5000
D = 128
N_EDGES = 320000
NPAD = 5120
LANES = 128
EPAD = 327680
NB = 2560
NW = 32
NBT = NB // NW
ROWS_PER_TILE = NPAD // 16


def _positional_encoding(n, d):
    position = np.arange(0, n, dtype=np.float32)[:, None]
    div_term = np.exp(np.arange(0, d, 2).astype(np.float32) * -(math.log(10000.0) / d))
    enc = np.zeros((n, d), dtype=np.float32)
    enc[:, 0::2] = np.sin(position * div_term)
    enc[:, 1::2] = np.cos(position * div_term)
    return jnp.asarray(enc)


@functools.cache
def _build_sc_kernels():
    mesh = plsc.VectorSubcoreMesh(
        core_axis_name="c", subcore_axis_name="s", num_cores=2, num_subcores=16
    )

    @functools.partial(
        pl.kernel,
        out_type=jax.ShapeDtypeStruct((2, NPAD, D), jnp.float32),
        mesh=mesh,
        scratch_types=[
            pltpu.VMEM_SHARED((NPAD, D), jnp.float32),
            pltpu.VMEM((NBT, LANES), jnp.int32),
            pltpu.VMEM((LANES, D), jnp.float32),
            pltpu.VMEM((80, D), jnp.float32),
            pltpu.SemaphoreType.DMA,
        ],
    )
    def sc_degree(colb_hbm, ones_hbm, z_hbm, out_hbm, acc, coli, ones_v, z_v, sem):
        c = lax.axis_index("c")
        s = lax.axis_index("s")
        wid = s * 2 + c
        pltpu.sync_copy(colb_hbm.at[pl.ds(wid * NBT, NBT)], coli)
        pltpu.sync_copy(ones_hbm, ones_v)
        pltpu.sync_copy(z_hbm, z_v)
        for k in range(ROWS_PER_TILE // 80):
            pltpu.sync_copy(z_v, acc.at[pl.ds(s * ROWS_PER_TILE + k * 80, 80)])
        plsc.subcore_barrier()

        def body(j, carry):
            pltpu.sync_copy(ones_v, acc.at[coli.at[j]], add=True)
            return carry

        lax.fori_loop(0, NBT, body, 0)
        plsc.subcore_barrier()
        pltpu.sync_copy(
            acc.at[pl.ds(s * ROWS_PER_TILE, ROWS_PER_TILE)],
            out_hbm.at[c, pl.ds(s * ROWS_PER_TILE, ROWS_PER_TILE)],
        )

    @functools.partial(
        pl.kernel,
        out_type=jax.ShapeDtypeStruct((2, NPAD, D), jnp.float32),
        mesh=mesh,
        scratch_types=[
            pltpu.VMEM_SHARED((NPAD, D), jnp.float32),
            pltpu.VMEM((NBT, LANES), jnp.int32),
            pltpu.VMEM((NBT, LANES), jnp.int32),
            pltpu.VMEM((80, D), jnp.float32),
            pltpu.VMEM((LANES, D), jnp.float32),
            pltpu.VMEM((LANES, D), jnp.float32),
            pltpu.SemaphoreType.DMA,
            pltpu.SemaphoreType.DMA,
        ],
    )
    def sc_messages(xs_hbm, rowb_hbm, colb_hbm, z_hbm, out_hbm,
                    acc, rowi, coli, z_v, rb_a, rb_b, sem_a, sem_b):
        c = lax.axis_index("c")
        s = lax.axis_index("s")
        wid = s * 2 + c
        pltpu.sync_copy(rowb_hbm.at[pl.ds(wid * NBT, NBT)], rowi)
        pltpu.sync_copy(colb_hbm.at[pl.ds(wid * NBT, NBT)], coli)
        pltpu.sync_copy(z_hbm, z_v)
        for k in range(ROWS_PER_TILE // 80):
            pltpu.sync_copy(z_v, acc.at[pl.ds(s * ROWS_PER_TILE + k * 80, 80)])
        plsc.subcore_barrier()

        pltpu.async_copy(xs_hbm.at[rowi.at[0]], rb_a, sem_a)

        def body(t, carry):
            j0 = 2 * t
            j1 = 2 * t + 1
            pltpu.make_async_copy(xs_hbm.at[rowi.at[j0]], rb_a, sem_a).wait()
            pltpu.async_copy(xs_hbm.at[rowi.at[j1]], rb_b, sem_b)
            pltpu.sync_copy(rb_a, acc.at[coli.at[j0]], add=True)
            pltpu.make_async_copy(xs_hbm.at[rowi.at[j1]], rb_b, sem_b).wait()

            @pl.when(t < NBT // 2 - 1)
            def _():
                pltpu.async_copy(xs_hbm.at[rowi.at[j1 + 1]], rb_a, sem_a)

            pltpu.sync_copy(rb_b, acc.at[coli.at[j1]], add=True)
            return carry

        lax.fori_loop(0, NBT // 2, body, 0)
        plsc.subcore_barrier()
        pltpu.sync_copy(
            acc.at[pl.ds(s * ROWS_PER_TILE, ROWS_PER_TILE)],
            out_hbm.at[c, pl.ds(s * ROWS_PER_TILE, ROWS_PER_TILE)],
        )

    return sc_degree, sc_messages


def _tc_prep_body(x_ref, pe_ref, w1_ref, dp_ref, xs_ref, dis_ref):
    deg = dp_ref[0, : N_NODES, 0:1] + dp_ref[1, : N_NODES, 0:1] + 1.0
    dis = lax.rsqrt(deg)
    xw = jnp.dot(x_ref[...] + pe_ref[...], w1_ref[...],
                 preferred_element_type=jnp.float32)
    xs_ref[...] = dis * xw
    dis_ref[...] = dis


def _tc_mid_body(m_ref, xs_ref, dis_ref, b1_ref, g_ref, bt_ref, w2_ref, out_ref):
    msum = m_ref[0, : N_NODES, :] + m_ref[1, : N_NODES, :]
    h = dis_ref[...] * (msum + xs_ref[...]) + b1_ref[...]
    mu = jnp.mean(h, axis=0, keepdims=True)
    var = jnp.mean((h - mu) * (h - mu), axis=0, keepdims=True)
    hn = g_ref[...] * (h - mu) * lax.rsqrt(var + 1e-5) + bt_ref[...]
    r = jnp.maximum(hn, 0.0)
    out_ref[...] = dis_ref[...] * jnp.dot(r, w2_ref[...],
                                          preferred_element_type=jnp.float32)


def _tc_final_body(m_ref, xs_ref, dis_ref, b2_ref, out_ref):
    msum = m_ref[0, : N_NODES, :] + m_ref[1, : N_NODES, :]
    out_ref[...] = jnp.maximum(dis_ref[...] * (msum + xs_ref[...]) + b2_ref[...], 0.0)


def kernel(x, edge_index, W1, b1, gamma, beta, W2, b2):
    pe = _positional_encoding(N_NODES, D)
    ei = edge_index.astype(jnp.int32)
    pad = jnp.broadcast_to(
        jnp.array([[0], [NPAD - 1]], dtype=jnp.int32), (2, EPAD - N_EDGES)
    )
    eib = jnp.concatenate([ei, pad], axis=1).reshape(2, NB, LANES)
    rowb = eib[0]
    colb = eib[1]

    ones16 = jnp.ones((LANES, D), jnp.float32)
    z128 = jnp.zeros((80, D), jnp.float32)

    sc_degree, sc_messages = _build_sc_kernels()
    degparts = sc_degree(colb, ones16, z128)

    xs1, dis = pl.pallas_call(
        _tc_prep_body,
        out_shape=(
            jax.ShapeDtypeStruct((N_NODES, D), jnp.float32),
            jax.ShapeDtypeStruct((N_NODES, 1), jnp.float32),
        ),
    )(x, pe, W1, degparts)

    m1 = sc_messages(xs1, rowb, colb, z128)

    xs2 = pl.pallas_call(
        _tc_mid_body,
        out_shape=jax.ShapeDtypeStruct((N_NODES, D), jnp.float32),
    )(m1, xs1, dis, b1.reshape(1, D), gamma.reshape(1, D), beta.reshape(1, D), W2)

    m2 = sc_messages(xs2, rowb, colb, z128)

    out = pl.pallas_call(
        _tc_final_body,
        out_shape=jax.ShapeDtypeStruct((N_NODES, D), jnp.float32),
    )(m2, xs2, dis, b2.reshape(1, D))
    return out


# speedup vs baseline: 20.9740x; 2.3193x over previous
import functools
import math

import jax
import jax.numpy as jnp
import numpy as np
from jax import lax
from jax.experimental import pallas as pl
from jax.experimental.pallas import tpu as pltpu
from jax.experimental.pallas import tpu_sc as plsc

N_NODES = 5000
D = 128
N_EDGES = 320000
NPAD = 5120
LANES = 128
EPAD = 327680
NB = 2560
NW = 32
NBT = NB // NW
ROWS_PER_TILE = NPAD // 16


def _positional_encoding(n, d):
    position = np.arange(0, n, dtype=np.float32)[:, None]
    div_term = np.exp(np.arange(0, d, 2).astype(np.float32) * -(math.log(10000.0) / d))
    enc = np.zeros((n, d), dtype=np.float32)
    enc[:, 0::2] = np.sin(position * div_term)
    enc[:, 1::2] = np.cos(position * div_term)
    return jnp.asarray(enc)


@functools.cache
def _build_sc_kernels():
    mesh = plsc.VectorSubcoreMesh(
        core_axis_name="c", subcore_axis_name="s", num_cores=2, num_subcores=16
    )

    @functools.partial(
        pl.kernel,
        out_type=jax.ShapeDtypeStruct((2, NPAD, D), jnp.float32),
        mesh=mesh,
        scratch_types=[
            pltpu.VMEM_SHARED((NPAD, D), jnp.float32),
            pltpu.VMEM((NBT, LANES), jnp.int32),
            pltpu.VMEM((LANES, D), jnp.float32),
            pltpu.VMEM((80, D), jnp.float32),
            pltpu.SemaphoreType.DMA,
        ],
    )
    def sc_degree(colb_hbm, ones_hbm, z_hbm, out_hbm, acc, coli, ones_v, z_v, sem):
        c = lax.axis_index("c")
        s = lax.axis_index("s")
        wid = s * 2 + c
        pltpu.sync_copy(colb_hbm.at[pl.ds(wid * NBT, NBT)], coli)
        pltpu.sync_copy(ones_hbm, ones_v)
        pltpu.sync_copy(z_hbm, z_v)
        for k in range(ROWS_PER_TILE // 80):
            pltpu.sync_copy(z_v, acc.at[pl.ds(s * ROWS_PER_TILE + k * 80, 80)])
        plsc.subcore_barrier()

        def body(j, carry):
            pltpu.sync_copy(ones_v, acc.at[coli.at[j]], add=True)
            return carry

        lax.fori_loop(0, NBT, body, 0)
        plsc.subcore_barrier()
        pltpu.sync_copy(
            acc.at[pl.ds(s * ROWS_PER_TILE, ROWS_PER_TILE)],
            out_hbm.at[c, pl.ds(s * ROWS_PER_TILE, ROWS_PER_TILE)],
        )

    @functools.partial(
        pl.kernel,
        out_type=jax.ShapeDtypeStruct((2, NPAD, D), jnp.float32),
        mesh=mesh,
        scratch_types=[
            pltpu.VMEM_SHARED((NPAD, D), jnp.float32),
            pltpu.VMEM_SHARED((NPAD, D), jnp.float32),
            pltpu.VMEM((NBT, LANES), jnp.int32),
            pltpu.VMEM((NBT, LANES), jnp.int32),
            pltpu.VMEM((LANES, D), jnp.float32),
            pltpu.SemaphoreType.DMA,
        ],
    )
    def sc_messages(xs_hbm, rowb_hbm, colb_hbm, z_hbm, out_hbm,
                    acc, xs_sh, rowi, coli, rb_a, sem_a):
        c = lax.axis_index("c")
        s = lax.axis_index("s")
        wid = s * 2 + c
        pltpu.sync_copy(rowb_hbm.at[pl.ds(wid * NBT, NBT)], rowi)
        pltpu.sync_copy(colb_hbm.at[pl.ds(wid * NBT, NBT)], coli)
        base = s * ROWS_PER_TILE
        pltpu.sync_copy(z_hbm, rb_a.at[pl.ds(0, 80)])
        for k in range(ROWS_PER_TILE // 80):
            pltpu.sync_copy(rb_a.at[pl.ds(0, 80)], acc.at[pl.ds(base + k * 80, 80)])
        for off, nr in ((0, LANES), (LANES, LANES), (2 * LANES, 64)):
            pltpu.sync_copy(xs_hbm.at[pl.ds(base + off, nr)], rb_a.at[pl.ds(0, nr)])
            pltpu.sync_copy(rb_a.at[pl.ds(0, nr)], xs_sh.at[pl.ds(base + off, nr)])
        plsc.subcore_barrier()

        def body(j, carry):
            pltpu.async_copy(xs_sh.at[rowi.at[j]], rb_a, sem_a).wait()
            pltpu.sync_copy(rb_a, acc.at[coli.at[j]], add=True)
            return carry

        lax.fori_loop(0, NBT, body, 0)
        plsc.subcore_barrier()
        pltpu.sync_copy(
            acc.at[pl.ds(base, ROWS_PER_TILE)],
            out_hbm.at[c, pl.ds(base, ROWS_PER_TILE)],
        )

    return sc_degree, sc_messages


def _tc_prep_body(x_ref, pe_ref, w1_ref, dp_ref, xs_ref, dis_ref):
    deg = dp_ref[0, : N_NODES, 0:1] + dp_ref[1, : N_NODES, 0:1] + 1.0
    dis = lax.rsqrt(deg)
    xw = jnp.dot(x_ref[...] + pe_ref[...], w1_ref[...],
                 preferred_element_type=jnp.float32)
    xs_ref[: N_NODES, :] = dis * xw
    xs_ref[N_NODES:, :] = jnp.zeros((NPAD - N_NODES, D), jnp.float32)
    dis_ref[...] = dis


def _tc_mid_body(m_ref, xs_ref, dis_ref, b1_ref, g_ref, bt_ref, w2_ref, out_ref):
    msum = m_ref[0, : N_NODES, :] + m_ref[1, : N_NODES, :]
    h = dis_ref[...] * (msum + xs_ref[: N_NODES, :]) + b1_ref[...]
    mu = jnp.mean(h, axis=0, keepdims=True)
    var = jnp.mean((h - mu) * (h - mu), axis=0, keepdims=True)
    hn = g_ref[...] * (h - mu) * lax.rsqrt(var + 1e-5) + bt_ref[...]
    r = jnp.maximum(hn, 0.0)
    out_ref[: N_NODES, :] = dis_ref[...] * jnp.dot(
        r, w2_ref[...], preferred_element_type=jnp.float32)
    out_ref[N_NODES:, :] = jnp.zeros((NPAD - N_NODES, D), jnp.float32)


def _tc_final_body(m_ref, xs_ref, dis_ref, b2_ref, out_ref):
    msum = m_ref[0, : N_NODES, :] + m_ref[1, : N_NODES, :]
    out_ref[...] = jnp.maximum(
        dis_ref[...] * (msum + xs_ref[: N_NODES, :]) + b2_ref[...], 0.0)


def kernel(x, edge_index, W1, b1, gamma, beta, W2, b2):
    pe = _positional_encoding(N_NODES, D)
    ei = edge_index.astype(jnp.int32)
    pad = jnp.broadcast_to(
        jnp.array([[0], [NPAD - 1]], dtype=jnp.int32), (2, EPAD - N_EDGES)
    )
    eib = jnp.concatenate([ei, pad], axis=1).reshape(2, NB, LANES)
    rowb = eib[0]
    colb = eib[1]

    ones16 = jnp.ones((LANES, D), jnp.float32)
    z128 = jnp.zeros((80, D), jnp.float32)

    sc_degree, sc_messages = _build_sc_kernels()
    degparts = sc_degree(colb, ones16, z128)

    xs1, dis = pl.pallas_call(
        _tc_prep_body,
        out_shape=(
            jax.ShapeDtypeStruct((NPAD, D), jnp.float32),
            jax.ShapeDtypeStruct((N_NODES, 1), jnp.float32),
        ),
    )(x, pe, W1, degparts)

    m1 = sc_messages(xs1, rowb, colb, z128)

    xs2 = pl.pallas_call(
        _tc_mid_body,
        out_shape=jax.ShapeDtypeStruct((NPAD, D), jnp.float32),
    )(m1, xs1, dis, b1.reshape(1, D), gamma.reshape(1, D), beta.reshape(1, D), W2)

    m2 = sc_messages(xs2, rowb, colb, z128)

    out = pl.pallas_call(
        _tc_final_body,
        out_shape=jax.ShapeDtypeStruct((N_NODES, D), jnp.float32),
    )(m2, xs2, dis, b2.reshape(1, D))
    return out


# speedup vs baseline: 25.4243x; 1.2122x over previous
import functools
import math

import jax
import jax.numpy as jnp
import numpy as np
from jax import lax
from jax.experimental import pallas as pl
from jax.experimental.pallas import tpu as pltpu
from jax.experimental.pallas import tpu_sc as plsc

N_NODES = 5000
D = 128
N_EDGES = 320000
NPAD = 5120
LANES = 128
EPAD = 327680
NB = 2560
NW = 32
NBT = NB // NW
ROWS_PER_TILE = NPAD // 16


def _positional_encoding(n, d):
    position = np.arange(0, n, dtype=np.float32)[:, None]
    div_term = np.exp(np.arange(0, d, 2).astype(np.float32) * -(math.log(10000.0) / d))
    enc = np.zeros((n, d), dtype=np.float32)
    enc[:, 0::2] = np.sin(position * div_term)
    enc[:, 1::2] = np.cos(position * div_term)
    return jnp.asarray(enc)


@functools.cache
def _build_sc_kernels():
    mesh = plsc.VectorSubcoreMesh(
        core_axis_name="c", subcore_axis_name="s", num_cores=2, num_subcores=16
    )

    @functools.partial(
        pl.kernel,
        out_type=jax.ShapeDtypeStruct((2, NPAD, D), jnp.float32),
        mesh=mesh,
        scratch_types=[
            pltpu.VMEM_SHARED((NPAD, D), jnp.float32),
            pltpu.VMEM((NBT, LANES), jnp.int32),
            pltpu.VMEM((LANES, D), jnp.float32),
            pltpu.VMEM((80, D), jnp.float32),
            pltpu.SemaphoreType.DMA,
        ],
    )
    def sc_degree(colb_hbm, ones_hbm, z_hbm, out_hbm, acc, coli, ones_v, z_v, sem):
        c = lax.axis_index("c")
        s = lax.axis_index("s")
        wid = s * 2 + c
        pltpu.sync_copy(colb_hbm.at[pl.ds(wid * NBT, NBT)], coli)
        pltpu.sync_copy(ones_hbm, ones_v)
        pltpu.sync_copy(z_hbm, z_v)
        for k in range(ROWS_PER_TILE // 80):
            pltpu.sync_copy(z_v, acc.at[pl.ds(s * ROWS_PER_TILE + k * 80, 80)])
        plsc.subcore_barrier()

        def body(j, carry):
            pltpu.sync_copy(ones_v, acc.at[coli.at[j]], add=True)
            return carry

        lax.fori_loop(0, NBT, body, 0)
        plsc.subcore_barrier()
        pltpu.sync_copy(
            acc.at[pl.ds(s * ROWS_PER_TILE, ROWS_PER_TILE)],
            out_hbm.at[c, pl.ds(s * ROWS_PER_TILE, ROWS_PER_TILE)],
        )

    @functools.partial(
        pl.kernel,
        out_type=jax.ShapeDtypeStruct((2, NPAD, D), jnp.float32),
        mesh=mesh,
        scratch_types=[
            pltpu.VMEM_SHARED((NPAD, D), jnp.float32),
            pltpu.VMEM_SHARED((NPAD, D), jnp.float32),
            pltpu.VMEM((NBT, LANES), jnp.int32),
            pltpu.VMEM((2, LANES), jnp.int32),
            pltpu.VMEM((2, LANES), jnp.int32),
            pltpu.VMEM((LANES, D), jnp.float32),
            pltpu.VMEM((LANES, D), jnp.float32),
            pltpu.SemaphoreType.DMA,
            pltpu.SemaphoreType.DMA,
        ],
    )
    def sc_messages(xs_hbm, packed_hbm, z_hbm, out_hbm,
                    acc, xs_sh, packv, rowu, colu, rb_a, rb_b, sem_a, sem_b):
        c = lax.axis_index("c")
        s = lax.axis_index("s")
        wid = s * 2 + c
        pltpu.sync_copy(packed_hbm.at[pl.ds(wid * NBT, NBT)], packv)
        base = s * ROWS_PER_TILE
        pltpu.sync_copy(z_hbm, rb_a.at[pl.ds(0, 80)])
        for k in range(ROWS_PER_TILE // 80):
            pltpu.sync_copy(rb_a.at[pl.ds(0, 80)], acc.at[pl.ds(base + k * 80, 80)])
        for off, nr in ((0, LANES), (LANES, LANES), (2 * LANES, 64)):
            pltpu.sync_copy(xs_hbm.at[pl.ds(base + off, nr)], rb_a.at[pl.ds(0, nr)])
            pltpu.sync_copy(rb_a.at[pl.ds(0, nr)], xs_sh.at[pl.ds(base + off, nr)])
        plsc.subcore_barrier()

        def unpack(j, slot):
            for k in range(LANES // 16):
                v = packv[j, pl.ds(k * 16, 16)]
                rowu[slot, pl.ds(k * 16, 16)] = lax.bitwise_and(v, 0xFFFF)
                colu[slot, pl.ds(k * 16, 16)] = lax.shift_right_logical(v, 16)

        unpack(0, 0)
        pltpu.async_copy(xs_sh.at[rowu.at[0]], rb_a, sem_a)

        def body(t, carry):
            j0 = 2 * t
            j1 = 2 * t + 1
            unpack(j1, 1)
            pltpu.make_async_copy(xs_sh.at[rowu.at[0]], rb_a, sem_a).wait()
            pltpu.async_copy(xs_sh.at[rowu.at[1]], rb_b, sem_b)
            pltpu.sync_copy(rb_a, acc.at[colu.at[0]], add=True)

            @pl.when(t < NBT // 2 - 1)
            def _():
                unpack(j1 + 1, 0)
                pltpu.make_async_copy(xs_sh.at[rowu.at[1]], rb_b, sem_b).wait()
                pltpu.async_copy(xs_sh.at[rowu.at[0]], rb_a, sem_a)
                pltpu.sync_copy(rb_b, acc.at[colu.at[1]], add=True)

            @pl.when(t == NBT // 2 - 1)
            def _():
                pltpu.make_async_copy(xs_sh.at[rowu.at[1]], rb_b, sem_b).wait()
                pltpu.sync_copy(rb_b, acc.at[colu.at[1]], add=True)

            return carry

        lax.fori_loop(0, NBT // 2, body, 0)
        plsc.subcore_barrier()
        pltpu.sync_copy(
            acc.at[pl.ds(base, ROWS_PER_TILE)],
            out_hbm.at[c, pl.ds(base, ROWS_PER_TILE)],
        )

    return sc_degree, sc_messages


def _tc_prep_body(x_ref, pe_ref, w1_ref, dp_ref, xs_ref, dis_ref):
    deg = dp_ref[0, : N_NODES, 0:1] + dp_ref[1, : N_NODES, 0:1] + 1.0
    dis = lax.rsqrt(deg)
    xw = jnp.dot(x_ref[...] + pe_ref[...], w1_ref[...],
                 preferred_element_type=jnp.float32)
    xs_ref[: N_NODES, :] = dis * xw
    xs_ref[N_NODES:, :] = jnp.zeros((NPAD - N_NODES, D), jnp.float32)
    dis_ref[...] = dis


def _tc_mid_body(m_ref, xs_ref, dis_ref, b1_ref, g_ref, bt_ref, w2_ref, out_ref):
    msum = m_ref[0, : N_NODES, :] + m_ref[1, : N_NODES, :]
    h = dis_ref[...] * (msum + xs_ref[: N_NODES, :]) + b1_ref[...]
    mu = jnp.mean(h, axis=0, keepdims=True)
    var = jnp.mean((h - mu) * (h - mu), axis=0, keepdims=True)
    hn = g_ref[...] * (h - mu) * lax.rsqrt(var + 1e-5) + bt_ref[...]
    r = jnp.maximum(hn, 0.0)
    out_ref[: N_NODES, :] = dis_ref[...] * jnp.dot(
        r, w2_ref[...], preferred_element_type=jnp.float32)
    out_ref[N_NODES:, :] = jnp.zeros((NPAD - N_NODES, D), jnp.float32)


def _tc_final_body(m_ref, xs_ref, dis_ref, b2_ref, out_ref):
    msum = m_ref[0, : N_NODES, :] + m_ref[1, : N_NODES, :]
    out_ref[...] = jnp.maximum(
        dis_ref[...] * (msum + xs_ref[: N_NODES, :]) + b2_ref[...], 0.0)


def kernel(x, edge_index, W1, b1, gamma, beta, W2, b2):
    pe = _positional_encoding(N_NODES, D)
    ei = edge_index.astype(jnp.int32)
    pad = jnp.broadcast_to(
        jnp.array([[0], [NPAD - 1]], dtype=jnp.int32), (2, EPAD - N_EDGES)
    )
    eib = jnp.concatenate([ei, pad], axis=1).reshape(2, NB, LANES)
    rowb = eib[0]
    colb = eib[1]
    packed = rowb | (colb << 16)

    ones16 = jnp.ones((LANES, D), jnp.float32)
    z128 = jnp.zeros((80, D), jnp.float32)

    sc_degree, sc_messages = _build_sc_kernels()
    degparts = sc_degree(colb, ones16, z128)

    xs1, dis = pl.pallas_call(
        _tc_prep_body,
        out_shape=(
            jax.ShapeDtypeStruct((NPAD, D), jnp.float32),
            jax.ShapeDtypeStruct((N_NODES, 1), jnp.float32),
        ),
    )(x, pe, W1, degparts)

    m1 = sc_messages(xs1, packed, z128)

    xs2 = pl.pallas_call(
        _tc_mid_body,
        out_shape=jax.ShapeDtypeStruct((NPAD, D), jnp.float32),
    )(m1, xs1, dis, b1.reshape(1, D), gamma.reshape(1, D), beta.reshape(1, D), W2)

    m2 = sc_messages(xs2, packed, z128)

    out = pl.pallas_call(
        _tc_final_body,
        out_shape=jax.ShapeDtypeStruct((N_NODES, D), jnp.float32),
    )(m2, xs2, dis, b2.reshape(1, D))
    return out
